# Initial kernel scaffold; baseline (speedup 1.0000x reference)
import jax
import jax.numpy as jnp
from jax.experimental import pallas as pl


def kernel(data, edge_index, w_pre1, b_pre1, w_pre2, b_pre2, w_pre3, b_pre3, w_pre4, b_pre4, w_post1, b_post1, w_post2, b_post2, gin0_w1, gin0_b1, gin0_gamma, gin0_beta, gin0_w2, gin0_b2, gin1_w1, gin1_b1, gin1_gamma, gin1_beta, gin1_w2, gin1_b2):
    raise NotImplementedError("write your pallas kernel here")


# speedup vs baseline: 2.7092x; 2.7092x over previous
import functools
import jax
import jax.numpy as jnp
from jax import lax
from jax.experimental import pallas as pl
from jax.experimental.pallas import tpu as pltpu
from jax.experimental.pallas import tpu_sc as plsc

N = 10000
E = 320000
D = 128
NC = 2
NS = 16
NW = NC * NS
CHUNK = 128
CPT = 80
E_PAD = NW * CPT * CHUNK
N_PAD = 10112
RPT = N_PAD // NS
BN_EPS = 1e-5


@functools.cache
def _make_sc_segsum():
    mesh = plsc.VectorSubcoreMesh(core_axis_name="c", subcore_axis_name="s",
                                  num_cores=NC, num_subcores=NS)

    @functools.partial(
        pl.kernel,
        out_type=jax.ShapeDtypeStruct((NC, N_PAD, D), jnp.float32),
        mesh=mesh,
        scratch_types=[
            pltpu.VMEM((CPT, CHUNK), jnp.int32),
            pltpu.VMEM((CPT, CHUNK), jnp.int32),
            pltpu.VMEM((CHUNK, D), jnp.float32),
            pltpu.VMEM_SHARED((N_PAD, D), jnp.float32),
            pltpu.SemaphoreType.DMA,
        ],
    )
    def sc_segsum(x_hbm, src_hbm, dst_hbm, zero_hbm, out_hbm,
                  src_v, dst_v, rows_v, acc, sem):
        c = lax.axis_index("c")
        s = lax.axis_index("s")
        wid = s * NC + c
        pltpu.sync_copy(zero_hbm.at[pl.ds(s * RPT, RPT)],
                        acc.at[pl.ds(s * RPT, RPT)])
        pltpu.sync_copy(src_hbm.at[wid], src_v)
        pltpu.sync_copy(dst_hbm.at[wid], dst_v)
        plsc.subcore_barrier()

        @pl.loop(0, CPT)
        def _chunk(j):
            pltpu.async_copy(x_hbm.at[src_v.at[j]], rows_v, sem).wait()
            pltpu.sync_copy(rows_v, acc.at[dst_v.at[j]], add=True)

        plsc.subcore_barrier()
        pltpu.sync_copy(acc.at[pl.ds(s * RPT, RPT)],
                        out_hbm.at[c, pl.ds(s * RPT, RPT)])

    return sc_segsum


def _sc_segsum(x, src, dst, zeros_pad):
    return _make_sc_segsum()(x, src, dst, zeros_pad)


BL = 1000


def _pre_body(d_ref, w1p_ref, b1_ref, w2_ref, b2_ref, w3p_ref, b3_ref,
              w4_ref, b4_ref, o_ref):
    d = d_ref[...]
    a = jnp.maximum(jnp.dot(d, w1p_ref[...], preferred_element_type=jnp.float32)
                    + b1_ref[...], 0.0)
    x2 = jnp.maximum(jnp.dot(a, w2_ref[...], preferred_element_type=jnp.float32)
                     + b2_ref[...], 0.0)
    i1 = jnp.maximum(jnp.dot(d, w3p_ref[...], preferred_element_type=jnp.float32)
                     + b3_ref[...], 0.0)
    i2 = jnp.maximum(jnp.dot(i1, w4_ref[...], preferred_element_type=jnp.float32)
                     + b4_ref[...], 0.0)
    o_ref[...] = jnp.concatenate((i2, x2), axis=1)


def _pre_mlp(data, w1p, b1, w2, b2, w3p, b3, w4, b4):
    grid = (N // BL,)
    return pl.pallas_call(
        _pre_body,
        grid=grid,
        in_specs=[
            pl.BlockSpec((BL, 1024), lambda i: (i, 0)),
            pl.BlockSpec((1024, 16), lambda i: (0, 0)),
            pl.BlockSpec((1, 16), lambda i: (0, 0)),
            pl.BlockSpec((16, 64), lambda i: (0, 0)),
            pl.BlockSpec((1, 64), lambda i: (0, 0)),
            pl.BlockSpec((1024, 256), lambda i: (0, 0)),
            pl.BlockSpec((1, 256), lambda i: (0, 0)),
            pl.BlockSpec((256, 64), lambda i: (0, 0)),
            pl.BlockSpec((1, 64), lambda i: (0, 0)),
        ],
        out_specs=pl.BlockSpec((BL, D), lambda i: (i, 0)),
        out_shape=jax.ShapeDtypeStruct((N, D), jnp.float32),
    )(data, w1p, b1, w2, b2, w3p, b3, w4, b4)


def _gin_in_body(x_ref, p0_ref, p1_ref, w1_ref, b1_ref, t_ref, st_ref):
    h = x_ref[...] + p0_ref[...] + p1_ref[...]
    t = jnp.dot(h, w1_ref[...], preferred_element_type=jnp.float32) + b1_ref[...]
    t_ref[...] = t

    @pl.when(pl.program_id(0) == 0)
    def _():
        st_ref[...] = jnp.zeros_like(st_ref)

    s1 = jnp.sum(t, axis=0, keepdims=True)
    s2 = jnp.sum(t * t, axis=0, keepdims=True)
    st_ref[...] += jnp.concatenate((s1, s2, jnp.zeros((6, D), jnp.float32)), axis=0)


def _gin_in(x, p0, p1, w1, b1):
    grid = (N // BL,)
    return pl.pallas_call(
        _gin_in_body,
        grid=grid,
        in_specs=[
            pl.BlockSpec((BL, D), lambda i: (i, 0)),
            pl.BlockSpec((BL, D), lambda i: (i, 0)),
            pl.BlockSpec((BL, D), lambda i: (i, 0)),
            pl.BlockSpec((D, D), lambda i: (0, 0)),
            pl.BlockSpec((1, D), lambda i: (0, 0)),
        ],
        out_specs=[
            pl.BlockSpec((BL, D), lambda i: (i, 0)),
            pl.BlockSpec((8, D), lambda i: (0, 0)),
        ],
        out_shape=[
            jax.ShapeDtypeStruct((N, D), jnp.float32),
            jax.ShapeDtypeStruct((8, D), jnp.float32),
        ],
    )(x, p0, p1, w1, b1)


def _bn(t, st_ref, gamma_ref, beta_ref):
    mean = st_ref[0:1, :] * (1.0 / N)
    var = st_ref[1:2, :] * (1.0 / N) - mean * mean
    inv = lax.rsqrt(var + BN_EPS)
    return (t - mean) * inv * gamma_ref[...] + beta_ref[...]


def _gin_out_body(t_ref, st_ref, g_ref, be_ref, w2_ref, b2_ref, r_ref, o_ref):
    tn = jnp.maximum(_bn(t_ref[...], st_ref, g_ref, be_ref), 0.0)
    o_ref[...] = (jnp.dot(tn, w2_ref[...], preferred_element_type=jnp.float32)
                  + b2_ref[...] + r_ref[...])


def _gin_out(t, st, gamma, beta, w2, b2, res):
    grid = (N // BL,)
    return pl.pallas_call(
        _gin_out_body,
        grid=grid,
        in_specs=[
            pl.BlockSpec((BL, D), lambda i: (i, 0)),
            pl.BlockSpec((8, D), lambda i: (0, 0)),
            pl.BlockSpec((1, D), lambda i: (0, 0)),
            pl.BlockSpec((1, D), lambda i: (0, 0)),
            pl.BlockSpec((D, D), lambda i: (0, 0)),
            pl.BlockSpec((1, D), lambda i: (0, 0)),
            pl.BlockSpec((BL, D), lambda i: (i, 0)),
        ],
        out_specs=pl.BlockSpec((BL, D), lambda i: (i, 0)),
        out_shape=jax.ShapeDtypeStruct((N, D), jnp.float32),
    )(t, st, gamma, beta, w2, b2, res)


def _final_body(t_ref, st_ref, g_ref, be_ref, w2_ref, b2_ref, r0_ref, r1_ref,
                wp1_ref, bp1_ref, wp2_ref, bp2_ref, o_ref):
    tn = jnp.maximum(_bn(t_ref[...], st_ref, g_ref, be_ref), 0.0)
    g1 = (jnp.dot(tn, w2_ref[...], preferred_element_type=jnp.float32)
          + b2_ref[...] + r0_ref[...] + r1_ref[...])
    a = jnp.maximum(jnp.dot(g1, wp1_ref[...], preferred_element_type=jnp.float32)
                    + bp1_ref[...], 0.0)
    o = jnp.dot(a, wp2_ref[...], preferred_element_type=jnp.float32) + bp2_ref[...]
    m = jnp.max(o, axis=1, keepdims=True)
    z = o - m
    lse = jnp.log(jnp.sum(jnp.exp(z), axis=1, keepdims=True))
    o_ref[...] = z - lse


def _final(t, st, gamma, beta, w2, b2, res0, res1, wp1, bp1, wp2p, bp2p):
    grid = (N // BL,)
    return pl.pallas_call(
        _final_body,
        grid=grid,
        in_specs=[
            pl.BlockSpec((BL, D), lambda i: (i, 0)),
            pl.BlockSpec((8, D), lambda i: (0, 0)),
            pl.BlockSpec((1, D), lambda i: (0, 0)),
            pl.BlockSpec((1, D), lambda i: (0, 0)),
            pl.BlockSpec((D, D), lambda i: (0, 0)),
            pl.BlockSpec((1, D), lambda i: (0, 0)),
            pl.BlockSpec((BL, D), lambda i: (i, 0)),
            pl.BlockSpec((BL, D), lambda i: (i, 0)),
            pl.BlockSpec((D, 32), lambda i: (0, 0)),
            pl.BlockSpec((1, 32), lambda i: (0, 0)),
            pl.BlockSpec((32, D), lambda i: (0, 0)),
            pl.BlockSpec((1, D), lambda i: (0, 0)),
        ],
        out_specs=pl.BlockSpec((BL, D), lambda i: (i, 0)),
        out_shape=jax.ShapeDtypeStruct((N, D), jnp.float32),
    )(t, st, gamma, beta, w2, b2, res0, res1, wp1, bp1, wp2p, bp2p)


def kernel(data, edge_index,
           w_pre1, b_pre1, w_pre2, b_pre2, w_pre3, b_pre3, w_pre4, b_pre4,
           w_post1, b_post1, w_post2, b_post2,
           gin0_w1, gin0_b1, gin0_gamma, gin0_beta, gin0_w2, gin0_b2,
           gin1_w1, gin1_b1, gin1_gamma, gin1_beta, gin1_w2, gin1_b2):
    f32 = jnp.float32
    w1p = jnp.zeros((1024, 16), f32).at[1022:, :].set(w_pre1)
    w3p = jnp.zeros((1024, 256), f32).at[:1022, :].set(w_pre3)
    wp2p = jnp.zeros((32, D), f32).at[:, :7].set(w_post2)
    bp2p = jnp.full((D,), -1e30, f32).at[:7].set(b_post2).reshape(1, D)

    row = lambda b: b.reshape(1, -1)

    new_x = _pre_mlp(data, w1p, row(b_pre1), w_pre2, row(b_pre2),
                     w3p, row(b_pre3), w_pre4, row(b_pre4))

    pad = E_PAD - E
    src = jnp.concatenate((edge_index[0], jnp.zeros((pad,), jnp.int32)))
    dst = jnp.concatenate((edge_index[1], jnp.full((pad,), N, jnp.int32)))
    src = src.reshape(NW, CPT, CHUNK)
    dst = dst.reshape(NW, CPT, CHUNK)
    zeros_pad = jnp.zeros((N_PAD, D), f32)

    parts0 = _sc_segsum(new_x, src, dst, zeros_pad)
    t0, st0 = _gin_in(new_x, parts0[0, :N], parts0[1, :N], gin0_w1, row(gin0_b1))
    g0 = _gin_out(t0, st0, row(gin0_gamma), row(gin0_beta), gin0_w2,
                  row(gin0_b2), new_x)

    parts1 = _sc_segsum(g0, src, dst, zeros_pad)
    t1, st1 = _gin_in(g0, parts1[0, :N], parts1[1, :N], gin1_w1, row(gin1_b1))
    out = _final(t1, st1, row(gin1_gamma), row(gin1_beta), gin1_w2,
                 row(gin1_b2), g0, new_x, w_post1, row(b_post1), wp2p, bp2p)
    return out[:, :7]
